# baseline (speedup 1.0000x reference)
import functools
import numpy as np
import jax
import jax.numpy as jnp
from jax import lax
from jax.experimental import pallas as pl
from jax.experimental.pallas import tpu as pltpu
from jax.experimental.pallas import tpu_sc as plsc

B_TOTAL = 16384
D = 128
NW = 32
R = B_TOTAL // NW
_TINY = np.float32(np.finfo(np.float32).tiny)
_LN2 = np.float32(0.6931471805599453)

_R0 = (13, 15, 26, 6)
_R1 = (17, 29, 16, 24)


def _threefry_bits(cnt):
    k0 = jnp.uint32(0)
    k1 = jnp.uint32(42)
    k2 = k0 ^ k1 ^ jnp.uint32(0x1BD11BDA)

    def four_rounds(x0, x1, rots):
        for r in rots:
            x0 = x0 + x1
            x1 = (x1 << jnp.uint32(r)) | (x1 >> jnp.uint32(32 - r))
            x1 = x0 ^ x1
        return x0, x1

    x0 = jnp.full_like(cnt, k0)
    x1 = cnt + k1
    x0, x1 = four_rounds(x0, x1, _R0)
    x0 = x0 + k1
    x1 = x1 + (k2 + jnp.uint32(1))
    x0, x1 = four_rounds(x0, x1, _R1)
    x0 = x0 + k2
    x1 = x1 + (k0 + jnp.uint32(2))
    x0, x1 = four_rounds(x0, x1, _R0)
    x0 = x0 + k0
    x1 = x1 + (k1 + jnp.uint32(3))
    x0, x1 = four_rounds(x0, x1, _R1)
    x0 = x0 + k1
    x1 = x1 + (k2 + jnp.uint32(4))
    x0, x1 = four_rounds(x0, x1, _R0)
    x0 = x0 + k2
    x1 = x1 + (k0 + jnp.uint32(5))
    return x0 ^ x1


def _sc_log(x):
    bits = lax.bitcast_convert_type(x, jnp.int32)
    e = (bits >> 23) - 127
    m = lax.bitcast_convert_type(
        (bits & jnp.int32(0x007FFFFF)) | jnp.int32(0x3F800000), jnp.float32)
    big = m > jnp.float32(1.4142135)
    m = jnp.where(big, m * jnp.float32(0.5), m)
    ef = (e + jnp.where(big, jnp.int32(1), jnp.int32(0))).astype(jnp.float32)
    t = m - jnp.float32(1.0)
    s = t / (jnp.float32(2.0) + t)
    z = s * s
    p = jnp.float32(1.0) + z * (jnp.float32(1.0 / 3.0) + z * (
        jnp.float32(0.2) + z * (jnp.float32(1.0 / 7.0) + z * jnp.float32(1.0 / 9.0))))
    return ef * _LN2 + jnp.float32(2.0) * s * p


def _gumbel16(cnt):
    bits = _threefry_bits(cnt)
    fb = (bits >> jnp.uint32(9)) | jnp.uint32(0x3F800000)
    f = lax.bitcast_convert_type(fb, jnp.float32) - jnp.float32(1.0)
    u = jnp.maximum(_TINY, f + _TINY)
    return -_sc_log(-_sc_log(u))


def _rne_bf16(v):
    bits = lax.bitcast_convert_type(v, jnp.int32)
    r = bits + jnp.int32(0x7FFF) + ((bits >> 16) & jnp.int32(1))
    return lax.bitcast_convert_type(r & jnp.int32(-65536), jnp.float32)


def _sc_body(x_hbm, wt_hbm, b_hbm,
             act_hbm, lpi_hbm, neg_hbm, hp_hbm,
             xv, wv, bv, l0v, l1v, l2v, actv, lpiv, negv, hpv):
    wid = lax.axis_index("s") * 2 + lax.axis_index("c")
    base = wid * R
    pltpu.sync_copy(x_hbm.at[pl.ds(base, R)], xv)
    pltpu.sync_copy(wt_hbm, wv)
    pltpu.sync_copy(b_hbm, bv)

    lane = lax.iota(jnp.int32, 16)
    m15 = lane == 15
    def lanesum(v):
        return plsc.cumsum(v)

    wchunks = [[_rne_bf16(wv[j, pl.ds(c * 16, 16)]) for c in range(8)]
               for j in range(3)]
    bvec = bv[...]
    bs = [bvec[j] for j in range(3)]
    lrefs = (l0v, l1v, l2v)

    def mm_row(r, carry):
        accs = []
        xc = [_rne_bf16(xv[r, pl.ds(c * 16, 16)]) for c in range(8)]
        for j in range(3):
            acc = xc[0] * wchunks[j][0]
            for c in range(1, 8):
                acc = acc + xc[c] * wchunks[j][c]
            accs.append(acc)
        ridx = jnp.full((16,), 0, jnp.int32) + r
        for j in range(3):
            tot = lanesum(accs[j]) + bs[j]
            plsc.store_scatter(lrefs[j], [ridx], tot, mask=m15)
        return carry

    lax.fori_loop(0, R, mm_row, 0)

    def samp_group(gi, carry):
        off = gi * 16
        l0 = l0v[pl.ds(off, 16)]
        l1 = l1v[pl.ds(off, 16)]
        l2 = l2v[pl.ds(off, 16)]
        row3 = ((base + off) + lane) * 3
        g0 = _gumbel16(row3.astype(jnp.uint32))
        g1 = _gumbel16((row3 + 1).astype(jnp.uint32))
        g2 = _gumbel16((row3 + 2).astype(jnp.uint32))
        y0 = g0 + l0
        y1 = g1 + l1
        y2 = g2 + l2
        a = jnp.where(y1 > y0, jnp.int32(1), jnp.int32(0))
        a = jnp.where(y2 > jnp.maximum(y0, y1), jnp.int32(2), a)
        m = jnp.maximum(jnp.maximum(l0, l1), l2)
        e0 = jnp.exp(l0 - m)
        e1 = jnp.exp(l1 - m)
        e2 = jnp.exp(l2 - m)
        ls = _sc_log(e0 + e1 + e2)
        lp0 = (l0 - m) - ls
        lp1 = (l1 - m) - ls
        lp2 = (l2 - m) - ls
        lpi = jnp.where(a == 0, lp0, jnp.where(a == 1, lp1, lp2))
        hp = jnp.exp(lp1)
        neg = -_sc_log(hp)
        actv[pl.ds(off, 16)] = a
        lpiv[pl.ds(off, 16)] = lpi
        negv[pl.ds(off, 16)] = neg
        hpv[pl.ds(off, 16)] = hp
        return carry

    lax.fori_loop(0, R // 16, samp_group, 0)

    pltpu.sync_copy(actv, act_hbm.at[pl.ds(base, R)])
    pltpu.sync_copy(lpiv, lpi_hbm.at[pl.ds(base, R)])
    pltpu.sync_copy(negv, neg_hbm.at[pl.ds(base, R)])
    pltpu.sync_copy(hpv, hp_hbm.at[pl.ds(base, R)])


def kernel(x, W, b):
    wt = W.T
    b16 = jnp.pad(b, (0, 13))
    mesh = plsc.VectorSubcoreMesh(core_axis_name="c", subcore_axis_name="s")
    sc = functools.partial(
        pl.kernel,
        out_type=[
            jax.ShapeDtypeStruct((B_TOTAL,), jnp.int32),
            jax.ShapeDtypeStruct((B_TOTAL,), jnp.float32),
            jax.ShapeDtypeStruct((B_TOTAL,), jnp.float32),
            jax.ShapeDtypeStruct((B_TOTAL,), jnp.float32),
        ],
        mesh=mesh,
        compiler_params=pltpu.CompilerParams(needs_layout_passes=False),
        scratch_types=[
            pltpu.VMEM((R, D), jnp.float32),
            pltpu.VMEM((3, D), jnp.float32),
            pltpu.VMEM((16,), jnp.float32),
            pltpu.VMEM((R,), jnp.float32),
            pltpu.VMEM((R,), jnp.float32),
            pltpu.VMEM((R,), jnp.float32),
            pltpu.VMEM((R,), jnp.int32),
            pltpu.VMEM((R,), jnp.float32),
            pltpu.VMEM((R,), jnp.float32),
            pltpu.VMEM((R,), jnp.float32),
        ],
    )(_sc_body)
    act, lpi, neg, hp = sc(x, wt, b16)
    rs = lambda t: t.reshape(B_TOTAL, 1)
    return (rs(act), rs(lpi), rs(neg), rs(hp))


# speedup vs baseline: 1.1401x; 1.1401x over previous
import functools
import numpy as np
import jax
import jax.numpy as jnp
from jax import lax
from jax.experimental import pallas as pl
from jax.experimental.pallas import tpu as pltpu
from jax.experimental.pallas import tpu_sc as plsc

B_TOTAL = 16384
D = 128
NW = 32
R = B_TOTAL // NW
_TINY = np.float32(np.finfo(np.float32).tiny)
_LN2 = np.float32(0.6931471805599453)

_R0 = (13, 15, 26, 6)
_R1 = (17, 29, 16, 24)


def _threefry_bits(cnt):
    k0 = jnp.uint32(0)
    k1 = jnp.uint32(42)
    k2 = k0 ^ k1 ^ jnp.uint32(0x1BD11BDA)

    def four_rounds(x0, x1, rots):
        for r in rots:
            x0 = x0 + x1
            x1 = (x1 << jnp.uint32(r)) | (x1 >> jnp.uint32(32 - r))
            x1 = x0 ^ x1
        return x0, x1

    x0 = jnp.full_like(cnt, k0)
    x1 = cnt + k1
    x0, x1 = four_rounds(x0, x1, _R0)
    x0 = x0 + k1
    x1 = x1 + (k2 + jnp.uint32(1))
    x0, x1 = four_rounds(x0, x1, _R1)
    x0 = x0 + k2
    x1 = x1 + (k0 + jnp.uint32(2))
    x0, x1 = four_rounds(x0, x1, _R0)
    x0 = x0 + k0
    x1 = x1 + (k1 + jnp.uint32(3))
    x0, x1 = four_rounds(x0, x1, _R1)
    x0 = x0 + k1
    x1 = x1 + (k2 + jnp.uint32(4))
    x0, x1 = four_rounds(x0, x1, _R0)
    x0 = x0 + k2
    x1 = x1 + (k0 + jnp.uint32(5))
    return x0 ^ x1


def _sc_log(x):
    bits = lax.bitcast_convert_type(x, jnp.int32)
    e = (bits >> 23) - 127
    m = lax.bitcast_convert_type(
        (bits & jnp.int32(0x007FFFFF)) | jnp.int32(0x3F800000), jnp.float32)
    big = m > jnp.float32(1.4142135)
    m = jnp.where(big, m * jnp.float32(0.5), m)
    ef = (e + jnp.where(big, jnp.int32(1), jnp.int32(0))).astype(jnp.float32)
    t = m - jnp.float32(1.0)
    s = t / (jnp.float32(2.0) + t)
    z = s * s
    p = jnp.float32(1.0) + z * (jnp.float32(1.0 / 3.0) + z * (
        jnp.float32(0.2) + z * (jnp.float32(1.0 / 7.0) + z * jnp.float32(1.0 / 9.0))))
    return ef * _LN2 + jnp.float32(2.0) * s * p


def _gumbel16(cnt):
    bits = _threefry_bits(cnt)
    fb = (bits >> jnp.uint32(9)) | jnp.uint32(0x3F800000)
    f = lax.bitcast_convert_type(fb, jnp.float32) - jnp.float32(1.0)
    u = jnp.maximum(_TINY, f + _TINY)
    return -_sc_log(-_sc_log(u))


def _rne_bf16(v):
    bits = lax.bitcast_convert_type(v, jnp.int32)
    r = bits + jnp.int32(0x7FFF) + ((bits >> 16) & jnp.int32(1))
    return lax.bitcast_convert_type(r & jnp.int32(-65536), jnp.float32)


def _sc_body(x_hbm, wt_hbm, b_hbm,
             act_hbm, lpi_hbm, neg_hbm, hp_hbm,
             xv, wv, bv, l0v, l1v, l2v, actv, lpiv, negv, hpv):
    wid = lax.axis_index("s") * 2 + lax.axis_index("c")
    base = wid * R
    pltpu.sync_copy(x_hbm.at[pl.ds(base, R)], xv)
    pltpu.sync_copy(wt_hbm, wv)
    pltpu.sync_copy(b_hbm, bv)

    lane = lax.iota(jnp.int32, 16)
    m15 = lane == 15
    def lanesum(v):
        return plsc.cumsum(v)

    wchunks = [[_rne_bf16(wv[j, pl.ds(c * 16, 16)]) for c in range(8)]
               for j in range(3)]
    bvec = bv[...]
    bs = [bvec[j] for j in range(3)]
    lrefs = (l0v, l1v, l2v)

    @plsc.parallel_loop(0, R, 1, unroll=4)
    def mm_row(r):
        accs = []
        xc = [_rne_bf16(xv[r, pl.ds(c * 16, 16)]) for c in range(8)]
        for j in range(3):
            acc = xc[0] * wchunks[j][0]
            for c in range(1, 8):
                acc = acc + xc[c] * wchunks[j][c]
            accs.append(acc)
        ridx = jnp.full((16,), 0, jnp.int32) + r
        for j in range(3):
            tot = lanesum(accs[j]) + bs[j]
            plsc.store_scatter(lrefs[j], [ridx], tot, mask=m15)

    @plsc.parallel_loop(0, R // 16, 1, unroll=2)
    def samp_group(gi):
        off = gi * 16
        l0 = l0v[pl.ds(off, 16)]
        l1 = l1v[pl.ds(off, 16)]
        l2 = l2v[pl.ds(off, 16)]
        row3 = ((base + off) + lane) * 3
        g0 = _gumbel16(row3.astype(jnp.uint32))
        g1 = _gumbel16((row3 + 1).astype(jnp.uint32))
        g2 = _gumbel16((row3 + 2).astype(jnp.uint32))
        y0 = g0 + l0
        y1 = g1 + l1
        y2 = g2 + l2
        a = jnp.where(y1 > y0, jnp.int32(1), jnp.int32(0))
        a = jnp.where(y2 > jnp.maximum(y0, y1), jnp.int32(2), a)
        m = jnp.maximum(jnp.maximum(l0, l1), l2)
        e0 = jnp.exp(l0 - m)
        e1 = jnp.exp(l1 - m)
        e2 = jnp.exp(l2 - m)
        ls = _sc_log(e0 + e1 + e2)
        lp0 = (l0 - m) - ls
        lp1 = (l1 - m) - ls
        lp2 = (l2 - m) - ls
        lpi = jnp.where(a == 0, lp0, jnp.where(a == 1, lp1, lp2))
        hp = jnp.exp(lp1)
        neg = -_sc_log(hp)
        actv[pl.ds(off, 16)] = a
        lpiv[pl.ds(off, 16)] = lpi
        negv[pl.ds(off, 16)] = neg
        hpv[pl.ds(off, 16)] = hp

    pltpu.sync_copy(actv, act_hbm.at[pl.ds(base, R)])
    pltpu.sync_copy(lpiv, lpi_hbm.at[pl.ds(base, R)])
    pltpu.sync_copy(negv, neg_hbm.at[pl.ds(base, R)])
    pltpu.sync_copy(hpv, hp_hbm.at[pl.ds(base, R)])


def kernel(x, W, b):
    wt = W.T
    b16 = jnp.pad(b, (0, 13))
    mesh = plsc.VectorSubcoreMesh(core_axis_name="c", subcore_axis_name="s")
    sc = functools.partial(
        pl.kernel,
        out_type=[
            jax.ShapeDtypeStruct((B_TOTAL,), jnp.int32),
            jax.ShapeDtypeStruct((B_TOTAL,), jnp.float32),
            jax.ShapeDtypeStruct((B_TOTAL,), jnp.float32),
            jax.ShapeDtypeStruct((B_TOTAL,), jnp.float32),
        ],
        mesh=mesh,
        compiler_params=pltpu.CompilerParams(needs_layout_passes=False),
        scratch_types=[
            pltpu.VMEM((R, D), jnp.float32),
            pltpu.VMEM((3, D), jnp.float32),
            pltpu.VMEM((16,), jnp.float32),
            pltpu.VMEM((R,), jnp.float32),
            pltpu.VMEM((R,), jnp.float32),
            pltpu.VMEM((R,), jnp.float32),
            pltpu.VMEM((R,), jnp.int32),
            pltpu.VMEM((R,), jnp.float32),
            pltpu.VMEM((R,), jnp.float32),
            pltpu.VMEM((R,), jnp.float32),
        ],
    )(_sc_body)
    act, lpi, neg, hp = sc(x, wt, b16)
    rs = lambda t: t.reshape(B_TOTAL, 1)
    return (rs(act), rs(lpi), rs(neg), rs(hp))
